# Initial kernel scaffold; baseline (speedup 1.0000x reference)
import jax
import jax.numpy as jnp
from jax.experimental import pallas as pl


def kernel(x, edge_index, W, b):
    raise NotImplementedError("write your pallas kernel here")


# speedup vs baseline: 5.3617x; 5.3617x over previous
import jax
import jax.numpy as jnp
from jax import lax
from jax.experimental import pallas as pl
from jax.experimental.pallas import tpu as pltpu
from jax.experimental.pallas import tpu_sc as plsc

N_NODES = 10000
D = 256
HD = 128
E = 160000

NC, NS, L = 2, 16, 16
HALF = N_NODES // NC
CHUNK = 128
N_CHUNKS = 80
E_TILE = CHUNK * N_CHUNKS
E_PAD = NS * E_TILE
ROWS_T = 320
R_PAD = ROWS_T * NS
DUMMY = HALF
XPAD = 10240

_MESH = plsc.VectorSubcoreMesh(
    core_axis_name="c", subcore_axis_name="s", num_cores=NC, num_subcores=NS)


def _hist_body(dst_hbm, hbuf, dst_v, ones_v, z_v, hist_sh):
    c = lax.axis_index("c")
    s = lax.axis_index("s")
    base = c * HALF
    zvec = jnp.zeros((L,), jnp.float32)
    def zloop(i, carry):
        z_v[i, :] = zvec
        return carry
    lax.fori_loop(0, ROWS_T, zloop, 0)
    pltpu.sync_copy(z_v, hist_sh.at[pl.ds(s * ROWS_T, ROWS_T)])
    onerow = jnp.where(lax.iota(jnp.int32, L) == 0, 1.0, 0.0)
    def oloop(i, carry):
        ones_v[i, :] = onerow
        return carry
    lax.fori_loop(0, CHUNK, oloop, 0)
    plsc.subcore_barrier()
    def sloop(j, carry):
        pltpu.sync_copy(dst_hbm.at[s].at[j], dst_v)
        for v in range(CHUNK // L):
            sl = pl.ds(v * L, L)
            loc = dst_v[sl] - base
            ok = (loc >= 0) & (loc < HALF)
            dst_v[sl] = jnp.where(ok, loc, DUMMY)
        pltpu.sync_copy(ones_v, hist_sh.at[dst_v], add=True)
        return carry
    lax.fori_loop(0, N_CHUNKS, sloop, 0)
    plsc.subcore_barrier()
    pltpu.sync_copy(hist_sh.at[pl.ds(s * ROWS_T, ROWS_T)],
                    hbuf.at[c, pl.ds(s * ROWS_T, ROWS_T)])


_hist_call = pl.kernel(
    _hist_body,
    out_type=jax.ShapeDtypeStruct((NC, R_PAD, L), jnp.float32),
    mesh=_MESH,
    scratch_types=[
        pltpu.VMEM((CHUNK,), jnp.int32),
        pltpu.VMEM((CHUNK, L), jnp.float32),
        pltpu.VMEM((ROWS_T, L), jnp.float32),
        pltpu.VMEM_SHARED((R_PAD, L), jnp.float32),
    ],
)


def _scat_body(y2_hbm, src_hbm, dst_hbm, out2_hbm,
               si_v, sa_v, sb_v, da_v, db_v, rh0_v, rh1_v, sem0, sem1,
               acc2_sh):
    c = lax.axis_index("c")
    s = lax.axis_index("s")
    base = c * HALF
    pltpu.sync_copy(y2_hbm.at[pl.ds((base + s * ROWS_T) * 2, ROWS_T * 2)],
                    acc2_sh.at[pl.ds(s * ROWS_T * 2, ROWS_T * 2)])
    plsc.subcore_barrier()
    def gloop(j, carry):
        pltpu.sync_copy(src_hbm.at[s].at[j], si_v)
        for v in range(CHUNK // L):
            sl = pl.ds(v * L, L)
            s2 = si_v[sl] * 2
            sa_v[sl] = s2
            sb_v[sl] = s2 + 1
        pltpu.sync_copy(dst_hbm.at[s].at[j], si_v)
        for v in range(CHUNK // L):
            sl = pl.ds(v * L, L)
            loc = si_v[sl] - base
            ok = (loc >= 0) & (loc < HALF)
            loc2 = jnp.where(ok, loc, DUMMY) * 2
            da_v[sl] = loc2
            db_v[sl] = loc2 + 1
        ga = pltpu.async_copy(y2_hbm.at[sa_v], rh0_v, sem0)
        gb = pltpu.async_copy(y2_hbm.at[sb_v], rh1_v, sem1)
        ga.wait()
        pltpu.sync_copy(rh0_v, acc2_sh.at[da_v], add=True)
        gb.wait()
        pltpu.sync_copy(rh1_v, acc2_sh.at[db_v], add=True)
        return carry
    lax.fori_loop(0, N_CHUNKS, gloop, 0)
    plsc.subcore_barrier()
    pltpu.sync_copy(acc2_sh.at[pl.ds(s * ROWS_T * 2, ROWS_T * 2)],
                    out2_hbm.at[c, pl.ds(s * ROWS_T * 2, ROWS_T * 2)])


_scat_call = pl.kernel(
    _scat_body,
    out_type=jax.ShapeDtypeStruct((NC, R_PAD * 2, HD), jnp.float32),
    mesh=_MESH,
    scratch_types=[
        pltpu.VMEM((CHUNK,), jnp.int32),
        pltpu.VMEM((CHUNK,), jnp.int32),
        pltpu.VMEM((CHUNK,), jnp.int32),
        pltpu.VMEM((CHUNK,), jnp.int32),
        pltpu.VMEM((CHUNK,), jnp.int32),
        pltpu.VMEM((CHUNK, HD), jnp.float32),
        pltpu.VMEM((CHUNK, HD), jnp.float32),
        pltpu.SemaphoreType.DMA,
        pltpu.SemaphoreType.DMA,
        pltpu.VMEM_SHARED((R_PAD * 2, HD), jnp.float32),
    ],
)


def _ac_body(x_ref, w_ref, h_ref, y_ref, dis_ref):
    xw = jnp.dot(x_ref[...], w_ref[...], preferred_element_type=jnp.float32)
    dis = lax.rsqrt(1.0 + h_ref[...])
    y_ref[...] = xw * dis
    dis_ref[...] = dis


_ac_call = pl.pallas_call(
    _ac_body,
    grid=(XPAD // 1024,),
    in_specs=[
        pl.BlockSpec((1024, D), lambda i: (i, 0)),
        pl.BlockSpec((D, D), lambda i: (0, 0)),
        pl.BlockSpec((1024, 1), lambda i: (i, 0)),
    ],
    out_specs=[
        pl.BlockSpec((1024, D), lambda i: (i, 0)),
        pl.BlockSpec((1024, 1), lambda i: (i, 0)),
    ],
    out_shape=[
        jax.ShapeDtypeStruct((XPAD, D), jnp.float32),
        jax.ShapeDtypeStruct((XPAD, 1), jnp.float32),
    ],
)


def _fin_body(a_ref, dis_ref, b_ref, o_ref):
    o_ref[...] = jnp.maximum(a_ref[...] * dis_ref[...] + b_ref[...], 0.0)


_fin_call = pl.pallas_call(
    _fin_body,
    grid=(N_NODES // 1000,),
    in_specs=[
        pl.BlockSpec((1000, D), lambda i: (i, 0)),
        pl.BlockSpec((1000, 1), lambda i: (i, 0)),
        pl.BlockSpec((1, D), lambda i: (0, 0)),
    ],
    out_specs=pl.BlockSpec((1000, D), lambda i: (i, 0)),
    out_shape=jax.ShapeDtypeStruct((N_NODES, D), jnp.float32),
)


def kernel(x, edge_index, W, b):
    src = edge_index[0].astype(jnp.int32)
    dst = edge_index[1].astype(jnp.int32)
    npad = E_PAD - E
    src_p = jnp.concatenate(
        [src, jnp.zeros((npad,), jnp.int32)]).reshape(NS, N_CHUNKS, CHUNK)
    dst_p = jnp.concatenate(
        [dst, jnp.full((npad,), N_NODES, jnp.int32)]).reshape(NS, N_CHUNKS, CHUNK)
    x_p = jnp.pad(x, ((0, XPAD - N_NODES), (0, 0)))

    hbuf = _hist_call(dst_p)
    hcol = jnp.concatenate(
        [hbuf[0, :HALF, 0], hbuf[1, :HALF, 0],
         jnp.zeros((XPAD - N_NODES,), jnp.float32)]).reshape(XPAD, 1)
    y, dis = _ac_call(x_p, W, hcol)
    y2 = y.reshape(XPAD * 2, HD)
    out2 = _scat_call(y2, src_p, dst_p)
    accbuf = out2.reshape(NC, R_PAD, D)
    acc = jnp.concatenate([accbuf[0, :HALF], accbuf[1, :HALF]], axis=0)
    return _fin_call(acc, dis[:N_NODES], b.reshape(1, D))


# speedup vs baseline: 6.2293x; 1.1618x over previous
import jax
import jax.numpy as jnp
from jax import lax
from jax.experimental import pallas as pl
from jax.experimental.pallas import tpu as pltpu
from jax.experimental.pallas import tpu_sc as plsc

N_NODES = 10000
D = 256
HD = 128
E = 160000

NC, NS, L = 2, 16, 16
HALF = N_NODES // NC
E_TILE = 10240
E_PAD = NS * E_TILE
ROWS_T = 320
R_PAD = ROWS_T * NS
DUMMY = HALF
XPAD = 10240

CH_H = 128
NCH_H = E_TILE // CH_H
BAT_H = 8

CH_S = 64
NCH_S = E_TILE // CH_S
BAT_S = 16

_MESH = plsc.VectorSubcoreMesh(
    core_axis_name="c", subcore_axis_name="s", num_cores=NC, num_subcores=NS)


def _hist_body(dst_hbm, hbuf, raw_d, dd0, dd1, ones_v, z_v, sems0, sems1,
               hist_sh):
    c = lax.axis_index("c")
    s = lax.axis_index("s")
    base = c * HALF
    zvec = jnp.zeros((L,), jnp.float32)
    def zloop(i, carry):
        z_v[i, :] = zvec
        return carry
    lax.fori_loop(0, ROWS_T, zloop, 0)
    pltpu.sync_copy(z_v, hist_sh.at[pl.ds(s * ROWS_T, ROWS_T)])
    onerow = jnp.where(lax.iota(jnp.int32, L) == 0, 1.0, 0.0)
    def oloop(i, carry):
        ones_v[i, :] = onerow
        return carry
    lax.fori_loop(0, CH_H, oloop, 0)
    plsc.subcore_barrier()

    sets = ((dd0, sems0), (dd1, sems1))

    def wait_add(S):
        dd, sems = S
        pltpu.make_async_copy(ones_v, hist_sh.at[dd], sems).wait()

    def batch_body(b, carry):
        pltpu.sync_copy(dst_hbm.at[s, pl.ds(b * BAT_H, BAT_H)], raw_d)
        for u in range(BAT_H):
            dd, sems = S = sets[u % 2]
            if u >= 2:
                wait_add(S)
            else:
                @pl.when(b > 0)
                def _():
                    wait_add(S)
            for v in range(CH_H // L):
                sl = pl.ds(v * L, L)
                loc = raw_d[u, sl] - base
                ok = (loc >= 0) & (loc < HALF)
                dd[sl] = jnp.where(ok, loc, DUMMY)
            pltpu.async_copy(ones_v, hist_sh.at[dd], sems, add=True)
        return carry
    lax.fori_loop(0, NCH_H // BAT_H, batch_body, 0)
    wait_add(sets[0])
    wait_add(sets[1])
    plsc.subcore_barrier()
    pltpu.sync_copy(hist_sh.at[pl.ds(s * ROWS_T, ROWS_T)],
                    hbuf.at[c, pl.ds(s * ROWS_T, ROWS_T)])


_hist_call = pl.kernel(
    _hist_body,
    out_type=jax.ShapeDtypeStruct((NC, R_PAD, L), jnp.float32),
    mesh=_MESH,
    scratch_types=[
        pltpu.VMEM((BAT_H, CH_H), jnp.int32),
        pltpu.VMEM((CH_H,), jnp.int32),
        pltpu.VMEM((CH_H,), jnp.int32),
        pltpu.VMEM((CH_H, L), jnp.float32),
        pltpu.VMEM((ROWS_T, L), jnp.float32),
        pltpu.SemaphoreType.DMA,
        pltpu.SemaphoreType.DMA,
        pltpu.VMEM_SHARED((R_PAD, L), jnp.float32),
    ],
)


def _scat_body(y2_hbm, src_hbm, dst_hbm, out2_hbm,
               raw_s, raw_d,
               sa0, sb0, da0, db0, sa1, sb1, da1, db1,
               rh00, rh01, rh10, rh11,
               semg0, semg1, sems0, sems1,
               acc2_sh):
    c = lax.axis_index("c")
    s = lax.axis_index("s")
    base = c * HALF
    pltpu.sync_copy(y2_hbm.at[pl.ds((base + s * ROWS_T) * 2, ROWS_T * 2)],
                    acc2_sh.at[pl.ds(s * ROWS_T * 2, ROWS_T * 2)])
    plsc.subcore_barrier()

    sets = ((sa0, sb0, da0, db0, rh00, rh01, semg0, sems0),
            (sa1, sb1, da1, db1, rh10, rh11, semg1, sems1))

    def build(u, S):
        sa, sb, da, db = S[0], S[1], S[2], S[3]
        for v in range(CH_S // L):
            sl = pl.ds(v * L, L)
            s2 = raw_s[u, sl] * 2
            sa[sl] = s2
            sb[sl] = s2 + 1
            loc = raw_d[u, sl] - base
            ok = (loc >= 0) & (loc < HALF)
            l2 = jnp.where(ok, loc, DUMMY) * 2
            da[sl] = l2
            db[sl] = l2 + 1

    def fire_gather(S):
        pltpu.async_copy(y2_hbm.at[S[0]], S[4], S[6])
        pltpu.async_copy(y2_hbm.at[S[1]], S[5], S[6])

    def wait_gather(S):
        pltpu.make_async_copy(y2_hbm.at[S[0]], S[4], S[6]).wait()
        pltpu.make_async_copy(y2_hbm.at[S[1]], S[5], S[6]).wait()

    def fire_scatter(S):
        pltpu.async_copy(S[4], acc2_sh.at[S[2]], S[7], add=True)
        pltpu.async_copy(S[5], acc2_sh.at[S[3]], S[7], add=True)

    def wait_scatter(S):
        pltpu.make_async_copy(S[4], acc2_sh.at[S[2]], S[7]).wait()
        pltpu.make_async_copy(S[5], acc2_sh.at[S[3]], S[7]).wait()

    def batch_body(b, carry):
        pltpu.sync_copy(src_hbm.at[s, pl.ds(b * BAT_S, BAT_S)], raw_s)
        pltpu.sync_copy(dst_hbm.at[s, pl.ds(b * BAT_S, BAT_S)], raw_d)
        for u in range(BAT_S):
            S = sets[u % 2]
            T = sets[1 - u % 2]
            if u >= 2:
                wait_scatter(S)
            else:
                @pl.when(b > 0)
                def _():
                    wait_scatter(S)
            build(u, S)
            fire_gather(S)
            if u >= 1:
                wait_gather(T)
                fire_scatter(T)
            else:
                @pl.when(b > 0)
                def _():
                    wait_gather(T)
                    fire_scatter(T)
        return carry
    lax.fori_loop(0, NCH_S // BAT_S, batch_body, 0)
    wait_gather(sets[1])
    fire_scatter(sets[1])
    wait_scatter(sets[0])
    wait_scatter(sets[1])
    plsc.subcore_barrier()
    pltpu.sync_copy(acc2_sh.at[pl.ds(s * ROWS_T * 2, ROWS_T * 2)],
                    out2_hbm.at[c, pl.ds(s * ROWS_T * 2, ROWS_T * 2)])


_scat_call = pl.kernel(
    _scat_body,
    out_type=jax.ShapeDtypeStruct((NC, R_PAD * 2, HD), jnp.float32),
    mesh=_MESH,
    scratch_types=[
        pltpu.VMEM((BAT_S, CH_S), jnp.int32),
        pltpu.VMEM((BAT_S, CH_S), jnp.int32),
        pltpu.VMEM((CH_S,), jnp.int32),
        pltpu.VMEM((CH_S,), jnp.int32),
        pltpu.VMEM((CH_S,), jnp.int32),
        pltpu.VMEM((CH_S,), jnp.int32),
        pltpu.VMEM((CH_S,), jnp.int32),
        pltpu.VMEM((CH_S,), jnp.int32),
        pltpu.VMEM((CH_S,), jnp.int32),
        pltpu.VMEM((CH_S,), jnp.int32),
        pltpu.VMEM((CH_S, HD), jnp.float32),
        pltpu.VMEM((CH_S, HD), jnp.float32),
        pltpu.VMEM((CH_S, HD), jnp.float32),
        pltpu.VMEM((CH_S, HD), jnp.float32),
        pltpu.SemaphoreType.DMA,
        pltpu.SemaphoreType.DMA,
        pltpu.SemaphoreType.DMA,
        pltpu.SemaphoreType.DMA,
        pltpu.VMEM_SHARED((R_PAD * 2, HD), jnp.float32),
    ],
)


def _ac_body(x_ref, w_ref, h_ref, y_ref, dis_ref):
    xw = jnp.dot(x_ref[...], w_ref[...], preferred_element_type=jnp.float32)
    dis = lax.rsqrt(1.0 + h_ref[...])
    y_ref[...] = xw * dis
    dis_ref[...] = dis


_ac_call = pl.pallas_call(
    _ac_body,
    grid=(XPAD // 1024,),
    in_specs=[
        pl.BlockSpec((1024, D), lambda i: (i, 0)),
        pl.BlockSpec((D, D), lambda i: (0, 0)),
        pl.BlockSpec((1024, 1), lambda i: (i, 0)),
    ],
    out_specs=[
        pl.BlockSpec((1024, D), lambda i: (i, 0)),
        pl.BlockSpec((1024, 1), lambda i: (i, 0)),
    ],
    out_shape=[
        jax.ShapeDtypeStruct((XPAD, D), jnp.float32),
        jax.ShapeDtypeStruct((XPAD, 1), jnp.float32),
    ],
)


def _fin_body(a_ref, dis_ref, b_ref, o_ref):
    o_ref[...] = jnp.maximum(a_ref[...] * dis_ref[...] + b_ref[...], 0.0)


_fin_call = pl.pallas_call(
    _fin_body,
    grid=(N_NODES // 1000,),
    in_specs=[
        pl.BlockSpec((1000, D), lambda i: (i, 0)),
        pl.BlockSpec((1000, 1), lambda i: (i, 0)),
        pl.BlockSpec((1, D), lambda i: (0, 0)),
    ],
    out_specs=pl.BlockSpec((1000, D), lambda i: (i, 0)),
    out_shape=jax.ShapeDtypeStruct((N_NODES, D), jnp.float32),
)


def kernel(x, edge_index, W, b):
    src = edge_index[0].astype(jnp.int32)
    dst = edge_index[1].astype(jnp.int32)
    npad = E_PAD - E
    src_flat = jnp.concatenate([src, jnp.zeros((npad,), jnp.int32)])
    dst_flat = jnp.concatenate([dst, jnp.full((npad,), N_NODES, jnp.int32)])
    src_s = src_flat.reshape(NS, NCH_S, CH_S)
    dst_s = dst_flat.reshape(NS, NCH_S, CH_S)
    dst_h = dst_flat.reshape(NS, NCH_H, CH_H)
    x_p = jnp.pad(x, ((0, XPAD - N_NODES), (0, 0)))

    hbuf = _hist_call(dst_h)
    hcol = jnp.concatenate(
        [hbuf[0, :HALF, 0], hbuf[1, :HALF, 0],
         jnp.zeros((XPAD - N_NODES,), jnp.float32)]).reshape(XPAD, 1)
    y, dis = _ac_call(x_p, W, hcol)
    y2 = y.reshape(XPAD * 2, HD)
    out2 = _scat_call(y2, src_s, dst_s)
    accbuf = out2.reshape(NC, R_PAD, D)
    acc = jnp.concatenate([accbuf[0, :HALF], accbuf[1, :HALF]], axis=0)
    return _fin_call(acc, dis[:N_NODES], b.reshape(1, D))
